# baseline (device time: 39876 ns/iter reference)
import jax
import jax.numpy as jnp
from jax import lax
from jax.experimental import pallas as pl
from jax.experimental.pallas import tpu as pltpu

N_DEV = 16
QUARTERS = (
    (15, 14, 13, 12),
    (11, 10, 9, 8),
    (7, 6, 5, 4),
    (3, 2, 1),
)
ALL_SLOTS = tuple(p for q in QUARTERS for p in q)


def kernel(x, Win0, Wout0, Win1, Wout1, Win2, Wout2):
    B, D = x.shape
    C = B // N_DEV

    def body(x_ref, win0_ref, wout0_ref, win1_ref, wout1_ref,
             win2_ref, wout2_ref, out_ref,
             xbuf0, xg, pbuf, pbf, crecv16, crecv32,
             send_a, recv_a, send_b, recv_b):
        i = lax.axis_index("i")
        f32, bf16 = jnp.float32, jnp.bfloat16

        barrier_sem = pltpu.get_barrier_semaphore()
        for o in range(1, N_DEV):
            pl.semaphore_signal(barrier_sem, inc=1,
                                device_id=((i + o) % N_DEV,),
                                device_id_type=pl.DeviceIdType.MESH)

        for q in range(N_DEV):
            xbuf0[pl.ds(q * C, C), :] = x_ref[
                pl.ds(((i - q) % N_DEV) * C, C), :]

        def compute_quarter(l, qi):
            lo = (3 - qi) * 4 * C
            n = 4 * C
            rows = pl.ds(lo, n)
            xr = xbuf0[rows, :] if l == 0 else xg[rows, :].astype(f32)
            win, wout = WINS[l], WOUTS[l]
            h = jnp.maximum(
                jnp.dot(xr, win[...], preferred_element_type=f32), 0.0)
            p = jnp.dot(h, wout[...], preferred_element_type=f32)
            pbuf[rows, :] = p
            if l < 2:
                pbf[rows, :] = p.astype(bf16)

        def rs_send(l, p):
            src, dst = (pbf, crecv16) if l < 2 else (pbuf, crecv32)
            d = pltpu.make_async_remote_copy(
                src_ref=src.at[pl.ds(p * C, C), :],
                dst_ref=dst.at[p],
                send_sem=send_a.at[p],
                recv_sem=recv_a.at[p],
                device_id=((i - p) % N_DEV,),
                device_id_type=pl.DeviceIdType.MESH,
            )
            d.start()
            return d

        def ag_send(o):
            d = pltpu.make_async_remote_copy(
                src_ref=xg.at[pl.ds(0, C), :],
                dst_ref=xg.at[pl.ds(o * C, C), :],
                send_sem=send_b.at[o],
                recv_sem=recv_b.at[o],
                device_id=((i + o) % N_DEV,),
                device_id_type=pl.DeviceIdType.MESH,
            )
            d.start()
            return d

        def ag_wait_recv(o):
            w = pltpu.make_async_remote_copy(
                src_ref=xg.at[pl.ds(o * C, C), :],
                dst_ref=xg.at[pl.ds(o * C, C), :],
                send_sem=send_b.at[o],
                recv_sem=recv_b.at[o],
                device_id=(i,),
                device_id_type=pl.DeviceIdType.MESH,
            )
            w.wait_recv()

        WINS = (win0_ref, win1_ref, win2_ref)
        WOUTS = (wout0_ref, wout1_ref, wout2_ref)

        ag_descs = []
        for l in range(3):
            rs_descs = {}
            for qi, batch in enumerate(QUARTERS):
                if l > 0:
                    for o in batch:
                        ag_wait_recv(o)
                compute_quarter(l, qi)
                if l == 0 and qi == 0:
                    pl.semaphore_wait(barrier_sem, N_DEV - 1)
                rs_descs.update({p: rs_send(l, p) for p in batch})
            for d in ag_descs:
                d.wait_send()

            crecv = crecv16 if l < 2 else crecv32
            acc = pbuf[pl.ds(0, C), :]
            for p in ALL_SLOTS:
                rs_descs[p].wait_recv()
                acc = acc + crecv[p].astype(f32)
            for d in rs_descs.values():
                d.wait_send()

            if l < 2:
                xg[pl.ds(0, C), :] = acc.astype(bf16)
                ag_descs = [ag_send(o) for o in ALL_SLOTS]
            else:
                out_ref[...] = acc

    return pl.pallas_call(
        body,
        out_shape=jax.ShapeDtypeStruct((C, D), jnp.float32),
        in_specs=[pl.BlockSpec(memory_space=pltpu.VMEM)] * 7,
        out_specs=pl.BlockSpec(memory_space=pltpu.VMEM),
        scratch_shapes=[
            pltpu.VMEM((B, D), jnp.float32),
            pltpu.VMEM((B, D), jnp.bfloat16),
            pltpu.VMEM((B, D), jnp.float32),
            pltpu.VMEM((B, D), jnp.bfloat16),
            pltpu.VMEM((N_DEV, C, D), jnp.bfloat16),
            pltpu.VMEM((N_DEV, C, D), jnp.float32),
            pltpu.SemaphoreType.DMA((N_DEV,)),
            pltpu.SemaphoreType.DMA((N_DEV,)),
            pltpu.SemaphoreType.DMA((N_DEV,)),
            pltpu.SemaphoreType.DMA((N_DEV,)),
        ],
        compiler_params=pltpu.CompilerParams(collective_id=0),
    )(x, Win0, Wout0, Win1, Wout1, Win2, Wout2)


# device time: 38730 ns/iter; 1.0296x vs baseline; 1.0296x over previous
import jax
import jax.numpy as jnp
from jax import lax
from jax.experimental import pallas as pl
from jax.experimental.pallas import tpu as pltpu

N_DEV = 16
QUARTERS = (
    (15, 14, 13, 12),
    (11, 10, 9, 8),
    (7, 6, 5, 4),
    (3, 2, 1),
)
ALL_SLOTS = tuple(p for q in QUARTERS for p in q)


def kernel(x, Win0, Wout0, Win1, Wout1, Win2, Wout2):
    B, D = x.shape
    C = B // N_DEV

    def body(x_ref, win0_ref, wout0_ref, win1_ref, wout1_ref,
             win2_ref, wout2_ref, out_ref,
             xbuf0, xg, pbuf, pbf, crecv16, crecv2,
             send_a, recv_a, send_b, recv_b, send_c, recv_c):
        i = lax.axis_index("i")
        f32, bf16 = jnp.float32, jnp.bfloat16

        barrier_sem = pltpu.get_barrier_semaphore()
        for o in range(1, N_DEV):
            pl.semaphore_signal(barrier_sem, inc=1,
                                device_id=((i + o) % N_DEV,),
                                device_id_type=pl.DeviceIdType.MESH)

        for q in range(N_DEV):
            xbuf0[pl.ds(q * C, C), :] = x_ref[
                pl.ds(((i - q) % N_DEV) * C, C), :]

        def compute_quarter(l, qi):
            lo = (3 - qi) * 4 * C
            n = 4 * C
            rows = pl.ds(lo, n)
            xr = xbuf0[rows, :] if l == 0 else xg[rows, :].astype(f32)
            win, wout = WINS[l], WOUTS[l]
            h = jnp.maximum(
                jnp.dot(xr, win[...], preferred_element_type=f32), 0.0)
            p = jnp.dot(h, wout[...], preferred_element_type=f32)
            pbuf[rows, :] = p
            pbf[rows, :] = p.astype(bf16)

        def rs_send(l, p):
            dst = crecv16 if l < 2 else crecv2
            ss = send_a if l < 2 else send_c
            rs = recv_a if l < 2 else recv_c
            d = pltpu.make_async_remote_copy(
                src_ref=pbf.at[pl.ds(p * C, C), :],
                dst_ref=dst.at[p],
                send_sem=ss.at[p],
                recv_sem=rs.at[p],
                device_id=((i - p) % N_DEV,),
                device_id_type=pl.DeviceIdType.MESH,
            )
            d.start()
            return d

        def ag_send(o):
            d = pltpu.make_async_remote_copy(
                src_ref=xg.at[pl.ds(0, C), :],
                dst_ref=xg.at[pl.ds(o * C, C), :],
                send_sem=send_b.at[o],
                recv_sem=recv_b.at[o],
                device_id=((i + o) % N_DEV,),
                device_id_type=pl.DeviceIdType.MESH,
            )
            d.start()
            return d

        def ag_wait_recv(o):
            w = pltpu.make_async_remote_copy(
                src_ref=xg.at[pl.ds(o * C, C), :],
                dst_ref=xg.at[pl.ds(o * C, C), :],
                send_sem=send_b.at[o],
                recv_sem=recv_b.at[o],
                device_id=(i,),
                device_id_type=pl.DeviceIdType.MESH,
            )
            w.wait_recv()

        WINS = (win0_ref, win1_ref, win2_ref)
        WOUTS = (wout0_ref, wout1_ref, wout2_ref)

        ag_descs = []
        for l in range(3):
            rs_descs = {}
            for qi, batch in enumerate(QUARTERS):
                if l > 0:
                    for o in batch:
                        ag_wait_recv(o)
                compute_quarter(l, qi)
                if l == 0 and qi == 0:
                    pl.semaphore_wait(barrier_sem, N_DEV - 1)
                rs_descs.update({p: rs_send(l, p) for p in batch})
            for d in ag_descs:
                d.wait_send()

            crecv = crecv16 if l < 2 else crecv2
            acc = pbuf[pl.ds(0, C), :]
            for p in ALL_SLOTS:
                rs_descs[p].wait_recv()
                acc = acc + crecv[p].astype(f32)
            for d in rs_descs.values():
                d.wait_send()

            if l < 2:
                xg[pl.ds(0, C), :] = acc.astype(bf16)
                ag_descs = [ag_send(o) for o in ALL_SLOTS]
            else:
                out_ref[...] = acc

    return pl.pallas_call(
        body,
        out_shape=jax.ShapeDtypeStruct((C, D), jnp.float32),
        in_specs=[pl.BlockSpec(memory_space=pltpu.VMEM)] * 7,
        out_specs=pl.BlockSpec(memory_space=pltpu.VMEM),
        scratch_shapes=[
            pltpu.VMEM((B, D), jnp.float32),
            pltpu.VMEM((B, D), jnp.bfloat16),
            pltpu.VMEM((B, D), jnp.float32),
            pltpu.VMEM((B, D), jnp.bfloat16),
            pltpu.VMEM((N_DEV, C, D), jnp.bfloat16),
            pltpu.VMEM((N_DEV, C, D), jnp.bfloat16),
            pltpu.SemaphoreType.DMA((N_DEV,)),
            pltpu.SemaphoreType.DMA((N_DEV,)),
            pltpu.SemaphoreType.DMA((N_DEV,)),
            pltpu.SemaphoreType.DMA((N_DEV,)),
            pltpu.SemaphoreType.DMA((N_DEV,)),
            pltpu.SemaphoreType.DMA((N_DEV,)),
        ],
        compiler_params=pltpu.CompilerParams(collective_id=0),
    )(x, Win0, Wout0, Win1, Wout1, Win2, Wout2)
